# baseline (device time: 273660 ns/iter reference)
import functools

import jax
import jax.numpy as jnp
from jax import lax
from jax.experimental import pallas as pl
from jax.experimental.pallas import tpu as pltpu

N_DEV = 8
SQ = 2048
DM = 1024
H_LOC = 8
DH = 128
CHUNK = SQ // N_DEV
SCALE = 0.08838834764831843


def _ring_allreduce(p_rot):

    def body(p_ref, out_ref, comm_ref, send_sems, rs_sems, ag_sems):
        pos = lax.axis_index("i")
        right = lax.rem(pos + 1, N_DEV)
        left = lax.rem(pos + N_DEV - 1, N_DEV)

        barrier = pltpu.get_barrier_semaphore()
        for nbr in (left, right):
            pl.semaphore_signal(
                barrier,
                inc=1,
                device_id=(nbr,),
                device_id_type=pl.DeviceIdType.MESH,
            )
        pl.semaphore_wait(barrier, 2)

        out_ref[...] = p_ref[...]

        for s in range(N_DEV - 1):
            send_slot = 7 - s
            recv_slot = 6 - s
            rdma = pltpu.make_async_remote_copy(
                src_ref=out_ref.at[send_slot],
                dst_ref=comm_ref.at[recv_slot],
                send_sem=send_sems.at[s],
                recv_sem=rs_sems.at[s],
                device_id=(right,),
                device_id_type=pl.DeviceIdType.MESH,
            )
            rdma.start()
            rdma.wait()
            out_ref[recv_slot] = out_ref[recv_slot] + comm_ref[recv_slot]

        for s in range(N_DEV - 1):
            send_slot = (N_DEV - s) % N_DEV
            recv_slot = 7 - s
            rdma = pltpu.make_async_remote_copy(
                src_ref=out_ref.at[send_slot],
                dst_ref=out_ref.at[recv_slot],
                send_sem=send_sems.at[N_DEV - 1 + s],
                recv_sem=ag_sems.at[s],
                device_id=(right,),
                device_id_type=pl.DeviceIdType.MESH,
            )
            rdma.start()
            rdma.wait()

    return pl.pallas_call(
        body,
        out_shape=jax.ShapeDtypeStruct((N_DEV, CHUNK, DM), jnp.float32),
        in_specs=[pl.BlockSpec(memory_space=pltpu.VMEM)],
        out_specs=pl.BlockSpec(memory_space=pltpu.VMEM),
        scratch_shapes=[
            pltpu.VMEM((N_DEV, CHUNK, DM), jnp.float32),
            pltpu.SemaphoreType.DMA((2 * (N_DEV - 1),)),
            pltpu.SemaphoreType.DMA((N_DEV - 1,)),
            pltpu.SemaphoreType.DMA((N_DEV - 1,)),
        ],
        compiler_params=pltpu.CompilerParams(collective_id=0),
    )(p_rot)


def kernel(x, Wq, K_ext, V_ext, Wo):
    pos = lax.axis_index("i")

    Wq_s = lax.dynamic_slice(Wq, (0, pos * H_LOC * DH), (DM, H_LOC * DH))
    Wo_s = lax.dynamic_slice(Wo, (pos * H_LOC * DH, 0), (H_LOC * DH, DM))

    xm = x[0]
    K = K_ext[0]
    V = V_ext[0]

    Q = (xm @ Wq_s).reshape(SQ, H_LOC, DH)

    def grp(t):
        return (
            t.reshape(8, 4, 64, H_LOC, DH)
            .transpose(1, 0, 2, 3, 4)
            .reshape(4, 512, H_LOC, DH)
        )

    Qg, Kg, Vg = grp(Q), grp(K), grp(V)
    s = jnp.einsum("gihd,gjhd->ghij", Qg, Kg) * SCALE
    w = jax.nn.softmax(s, axis=-1)
    ctxg = jnp.einsum("ghij,gjhd->gihd", w, Vg)
    ctx = (
        ctxg.reshape(4, 8, 64, H_LOC, DH)
        .transpose(1, 0, 2, 3, 4)
        .reshape(SQ, H_LOC * DH)
    )

    partial = ctx @ Wo_s

    p_rot = jnp.roll(
        partial.reshape(N_DEV, CHUNK, DM), shift=-(pos + 1), axis=0
    )
    out_rot = _ring_allreduce(p_rot)
    out = jnp.roll(out_rot, shift=pos + 1, axis=0).reshape(SQ, DM)
    return out[None]


# device time: 204934 ns/iter; 1.3354x vs baseline; 1.3354x over previous
import functools

import jax
import jax.numpy as jnp
from jax import lax
from jax.experimental import pallas as pl
from jax.experimental.pallas import tpu as pltpu

N_DEV = 8
SQ = 2048
DM = 1024
H_LOC = 8
DH = 128
CHUNK = SQ // N_DEV
SCALE = 0.08838834764831843


def _ring_allreduce_bidir(pR, pL):

    def body(pR_ref, pL_ref, outR_ref, outL_ref, commR_ref, commL_ref,
             send_sems, rs_sems, ag_sems):
        pos = lax.axis_index("i")
        right = lax.rem(pos + 1, N_DEV)
        left = lax.rem(pos + N_DEV - 1, N_DEV)

        barrier = pltpu.get_barrier_semaphore()
        for nbr in (left, right):
            pl.semaphore_signal(
                barrier,
                inc=1,
                device_id=(nbr,),
                device_id_type=pl.DeviceIdType.MESH,
            )
        pl.semaphore_wait(barrier, 2)

        outR_ref[...] = pR_ref[...]
        outL_ref[...] = pL_ref[...]

        def hop(s, out_ref, comm_ref, dev, sem_half):
            return pltpu.make_async_remote_copy(
                src_ref=out_ref.at[7 - s],
                dst_ref=comm_ref.at[6 - s],
                send_sem=send_sems.at[s, sem_half, 0],
                recv_sem=rs_sems.at[s, sem_half],
                device_id=(dev,),
                device_id_type=pl.DeviceIdType.MESH,
            )

        for s in range(N_DEV - 1):
            rdR = hop(s, outR_ref, commR_ref, right, 0)
            rdL = hop(s, outL_ref, commL_ref, left, 1)
            rdR.start()
            rdL.start()
            rdR.wait()
            rdL.wait()
            outR_ref[6 - s] = outR_ref[6 - s] + commR_ref[6 - s]
            outL_ref[6 - s] = outL_ref[6 - s] + commL_ref[6 - s]

        def ag_hop(s, out_ref, dev, sem_half):
            return pltpu.make_async_remote_copy(
                src_ref=out_ref.at[(N_DEV - s) % N_DEV],
                dst_ref=out_ref.at[7 - s],
                send_sem=send_sems.at[s, sem_half, 1],
                recv_sem=ag_sems.at[s, sem_half],
                device_id=(dev,),
                device_id_type=pl.DeviceIdType.MESH,
            )

        for s in range(N_DEV - 1):
            rdR = ag_hop(s, outR_ref, right, 0)
            rdL = ag_hop(s, outL_ref, left, 1)
            rdR.start()
            rdL.start()
            rdR.wait()
            rdL.wait()

    half = jax.ShapeDtypeStruct((N_DEV, CHUNK, DM // 2), jnp.float32)
    return pl.pallas_call(
        body,
        out_shape=(half, half),
        in_specs=[
            pl.BlockSpec(memory_space=pltpu.VMEM),
            pl.BlockSpec(memory_space=pltpu.VMEM),
        ],
        out_specs=(
            pl.BlockSpec(memory_space=pltpu.VMEM),
            pl.BlockSpec(memory_space=pltpu.VMEM),
        ),
        scratch_shapes=[
            pltpu.VMEM((N_DEV, CHUNK, DM // 2), jnp.float32),
            pltpu.VMEM((N_DEV, CHUNK, DM // 2), jnp.float32),
            pltpu.SemaphoreType.DMA((N_DEV - 1, 2, 2)),
            pltpu.SemaphoreType.DMA((N_DEV - 1, 2)),
            pltpu.SemaphoreType.DMA((N_DEV - 1, 2)),
        ],
        compiler_params=pltpu.CompilerParams(collective_id=0),
    )(pR, pL)


def kernel(x, Wq, K_ext, V_ext, Wo):
    pos = lax.axis_index("i")

    Wq_s = lax.dynamic_slice(Wq, (0, pos * H_LOC * DH), (DM, H_LOC * DH))
    Wo_s = lax.dynamic_slice(Wo, (pos * H_LOC * DH, 0), (H_LOC * DH, DM))

    xm = x[0]
    K = K_ext[0]
    V = V_ext[0]

    Q = (xm @ Wq_s).reshape(SQ, H_LOC, DH)

    def grp(t):
        return (
            t.reshape(8, 4, 64, H_LOC, DH)
            .transpose(1, 0, 2, 3, 4)
            .reshape(4, 512, H_LOC, DH)
        )

    Qg, Kg, Vg = grp(Q), grp(K), grp(V)
    s = jnp.einsum("gihd,gjhd->ghij", Qg, Kg) * SCALE
    w = jax.nn.softmax(s, axis=-1)
    ctxg = jnp.einsum("ghij,gjhd->gihd", w, Vg)
    ctx = (
        ctxg.reshape(4, 8, 64, H_LOC, DH)
        .transpose(1, 0, 2, 3, 4)
        .reshape(SQ, H_LOC * DH)
    )

    partial = ctx @ Wo_s

    p8 = partial.reshape(N_DEV, CHUNK, DM)
    pR = jnp.roll(p8[:, :, : DM // 2], shift=-(pos + 1), axis=0)
    pL = jnp.roll(p8[::-1, :, DM // 2 :], shift=pos, axis=0)
    outR_rot, outL_rot = _ring_allreduce_bidir(pR, pL)
    outR = jnp.roll(outR_rot, shift=pos + 1, axis=0)
    outL = jnp.roll(outL_rot, shift=-pos, axis=0)[::-1]
    out = jnp.concatenate([outR, outL], axis=2).reshape(SQ, DM)
    return out[None]


# device time: 195246 ns/iter; 1.4016x vs baseline; 1.0496x over previous
import jax
import jax.numpy as jnp
from jax import lax
from jax.experimental import pallas as pl
from jax.experimental.pallas import tpu as pltpu

N_DEV = 8
SQ = 2048
DM = 1024
HD = DM // 2
H_LOC = 8
DH = 128
CHUNK = SQ // N_DEV
SCALE = 0.08838834764831843


def _outproj_allreduce(ctx, Wo_s):

    def body(ctx_ref, wo_ref,
             outR_ref, outL_ref, commR_ref, commL_ref,
             send_sems, rs_sems, ag_sems):
        pos = lax.axis_index("i")
        right = lax.rem(pos + 1, N_DEV)
        left = lax.rem(pos + N_DEV - 1, N_DEV)

        def compute_chunk(m):
            c = lax.rem(pos - m + N_DEV, N_DEV)
            xr = ctx_ref[pl.ds(c * CHUNK, CHUNK), :]
            part = jnp.dot(xr, wo_ref[...],
                           preferred_element_type=jnp.float32)
            outR_ref[(7 - m) % N_DEV] = part[:, :HD]
            outL_ref[(m - 1) % N_DEV] = part[:, HD:]

        barrier = pltpu.get_barrier_semaphore()
        for nbr in (left, right):
            pl.semaphore_signal(
                barrier,
                inc=1,
                device_id=(nbr,),
                device_id_type=pl.DeviceIdType.MESH,
            )
        pl.semaphore_wait(barrier, 2)

        compute_chunk(0)

        def rs_hop(s, out_ref, comm_ref, dev, h):
            return pltpu.make_async_remote_copy(
                src_ref=out_ref.at[7 - s],
                dst_ref=comm_ref.at[6 - s],
                send_sem=send_sems.at[s, h, 0],
                recv_sem=rs_sems.at[s, h],
                device_id=(dev,),
                device_id_type=pl.DeviceIdType.MESH,
            )

        for s in range(N_DEV - 1):
            rdR = rs_hop(s, outR_ref, commR_ref, right, 0)
            rdL = rs_hop(s, outL_ref, commL_ref, left, 1)
            rdR.start()
            rdL.start()
            if s <= 2:
                compute_chunk(s + 1)
                compute_chunk(7 - s)
            elif s == 3:
                compute_chunk(4)
            rdR.wait()
            rdL.wait()
            outR_ref[6 - s] = outR_ref[6 - s] + commR_ref[6 - s]
            outL_ref[6 - s] = outL_ref[6 - s] + commL_ref[6 - s]

        def ag_hop(s, out_ref, dev, h):
            return pltpu.make_async_remote_copy(
                src_ref=out_ref.at[(N_DEV - s) % N_DEV],
                dst_ref=out_ref.at[7 - s],
                send_sem=send_sems.at[s, h, 1],
                recv_sem=ag_sems.at[s, h],
                device_id=(dev,),
                device_id_type=pl.DeviceIdType.MESH,
            )

        for s in range(N_DEV - 1):
            rdR = ag_hop(s, outR_ref, right, 0)
            rdL = ag_hop(s, outL_ref, left, 1)
            rdR.start()
            rdL.start()
            rdR.wait()
            rdL.wait()

    half = jax.ShapeDtypeStruct((N_DEV, CHUNK, HD), jnp.float32)
    vmem = pl.BlockSpec(memory_space=pltpu.VMEM)
    return pl.pallas_call(
        body,
        out_shape=(half, half),
        in_specs=[vmem, vmem],
        out_specs=(vmem, vmem),
        scratch_shapes=[
            pltpu.VMEM((N_DEV, CHUNK, HD), jnp.float32),
            pltpu.VMEM((N_DEV, CHUNK, HD), jnp.float32),
            pltpu.SemaphoreType.DMA((N_DEV - 1, 2, 2)),
            pltpu.SemaphoreType.DMA((N_DEV - 1, 2)),
            pltpu.SemaphoreType.DMA((N_DEV - 1, 2)),
        ],
        compiler_params=pltpu.CompilerParams(collective_id=0),
    )(ctx, Wo_s)


def kernel(x, Wq, K_ext, V_ext, Wo):
    pos = lax.axis_index("i")

    Wq_s = lax.dynamic_slice(Wq, (0, pos * H_LOC * DH), (DM, H_LOC * DH))
    Wo_s = lax.dynamic_slice(Wo, (pos * H_LOC * DH, 0), (H_LOC * DH, DM))

    Q = (x[0] @ Wq_s).reshape(SQ, H_LOC, DH)

    def grp(t):
        return (
            t.reshape(8, 4, 64, H_LOC, DH)
            .transpose(1, 0, 2, 3, 4)
            .reshape(4, 512, H_LOC, DH)
        )

    Qg, Kg, Vg = grp(Q), grp(K_ext[0]), grp(V_ext[0])
    s = jnp.einsum("gihd,gjhd->ghij", Qg, Kg) * SCALE
    w = jax.nn.softmax(s, axis=-1)
    ctxg = jnp.einsum("ghij,gjhd->gihd", w, Vg)
    ctx = (
        ctxg.reshape(4, 8, 64, H_LOC, DH)
        .transpose(1, 0, 2, 3, 4)
        .reshape(SQ, H_LOC * DH)
    )

    outR_rot, outL_rot = _outproj_allreduce(ctx, Wo_s)
    outR = jnp.roll(outR_rot, shift=pos + 1, axis=0)
    outL = jnp.roll(outL_rot, shift=-pos, axis=0)[::-1]
    out = jnp.concatenate([outR, outL], axis=2).reshape(SQ, DM)
    return out[None]


# device time: 179921 ns/iter; 1.5210x vs baseline; 1.0852x over previous
import jax
import jax.numpy as jnp
from jax import lax
from jax.experimental import pallas as pl
from jax.experimental.pallas import tpu as pltpu

N_DEV = 8
SQ = 2048
DM = 1024
HD = DM // 2
H_LOC = 8
DH = 128
CHUNK = SQ // N_DEV
SCALE = 0.08838834764831843


def _outproj_allreduce(ctx, Wo_s):

    def body(ctx_ref, wo_ref,
             outR_ref, outL_ref, commR_ref, commL_ref,
             send_sems, rs_sems, ag_sems):
        pos = lax.axis_index("i")
        right = lax.rem(pos + 1, N_DEV)
        left = lax.rem(pos + N_DEV - 1, N_DEV)

        def compute_chunk(m):
            c = lax.rem(pos - m + N_DEV, N_DEV)
            xr = ctx_ref[pl.ds(c * CHUNK, CHUNK), :]
            part = jnp.dot(xr, wo_ref[...],
                           preferred_element_type=jnp.float32)
            outR_ref[(7 - m) % N_DEV] = part[:, :HD]
            outL_ref[(m - 1) % N_DEV] = part[:, HD:]

        barrier = pltpu.get_barrier_semaphore()
        for nbr in (left, right):
            pl.semaphore_signal(
                barrier,
                inc=1,
                device_id=(nbr,),
                device_id_type=pl.DeviceIdType.MESH,
            )
        pl.semaphore_wait(barrier, 2)

        compute_chunk(0)

        def rs_hop(s, out_ref, comm_ref, dev, h):
            return pltpu.make_async_remote_copy(
                src_ref=out_ref.at[7 - s],
                dst_ref=comm_ref.at[6 - s],
                send_sem=send_sems.at[s, h, 0],
                recv_sem=rs_sems.at[s, h],
                device_id=(dev,),
                device_id_type=pl.DeviceIdType.MESH,
            )

        for s in range(N_DEV - 1):
            rdR = rs_hop(s, outR_ref, commR_ref, right, 0)
            rdL = rs_hop(s, outL_ref, commL_ref, left, 1)
            rdR.start()
            rdL.start()
            if s <= 2:
                compute_chunk(s + 1)
                compute_chunk(7 - s)
            elif s == 3:
                compute_chunk(4)
            rdR.wait()
            rdL.wait()
            outR_ref[6 - s] = outR_ref[6 - s] + commR_ref[6 - s]
            outL_ref[6 - s] = outL_ref[6 - s] + commL_ref[6 - s]

        def ag_hop(s, out_ref, dev, h):
            return pltpu.make_async_remote_copy(
                src_ref=out_ref.at[(N_DEV - s) % N_DEV],
                dst_ref=out_ref.at[7 - s],
                send_sem=send_sems.at[s, h, 1],
                recv_sem=ag_sems.at[s, h],
                device_id=(dev,),
                device_id_type=pl.DeviceIdType.MESH,
            )

        for s in range(N_DEV - 1):
            rdR = ag_hop(s, outR_ref, right, 0)
            rdL = ag_hop(s, outL_ref, left, 1)
            rdR.start()
            rdL.start()
            rdR.wait()
            rdL.wait()

    half = jax.ShapeDtypeStruct((N_DEV, CHUNK, HD), jnp.float32)
    vmem = pl.BlockSpec(memory_space=pltpu.VMEM)
    return pl.pallas_call(
        body,
        out_shape=(half, half),
        in_specs=[vmem, vmem],
        out_specs=(vmem, vmem),
        scratch_shapes=[
            pltpu.VMEM((N_DEV, CHUNK, HD), jnp.float32),
            pltpu.VMEM((N_DEV, CHUNK, HD), jnp.float32),
            pltpu.SemaphoreType.DMA((N_DEV - 1, 2, 2)),
            pltpu.SemaphoreType.DMA((N_DEV - 1, 2)),
            pltpu.SemaphoreType.DMA((N_DEV - 1, 2)),
        ],
        compiler_params=pltpu.CompilerParams(collective_id=0),
    )(ctx, Wo_s)


def _attention(Qt, Kt, Vt):

    def body(q_ref, k_ref, v_ref, o_ref):
        q = q_ref[0]
        k = k_ref[0]
        s = jax.lax.dot_general(
            q, k, (((2,), (2,)), ((0,), (0,))),
            preferred_element_type=jnp.float32,
        ) * SCALE
        s = s - jnp.max(s, axis=-1, keepdims=True)
        w = jnp.exp(s)
        w = w / jnp.sum(w, axis=-1, keepdims=True)
        o_ref[0] = jax.lax.dot_general(
            w, v_ref[0], (((2,), (1,)), ((0,), (0,))),
            preferred_element_type=jnp.float32,
        )

    blk = pl.BlockSpec((1, H_LOC, 512, DH), lambda g: (g, 0, 0, 0))
    return pl.pallas_call(
        body,
        grid=(4,),
        out_shape=jax.ShapeDtypeStruct((4, H_LOC, 512, DH), jnp.float32),
        in_specs=[blk, blk, blk],
        out_specs=blk,
    )(Qt, Kt, Vt)


def kernel(x, Wq, K_ext, V_ext, Wo):
    pos = lax.axis_index("i")

    Wq_s = lax.dynamic_slice(Wq, (0, pos * H_LOC * DH), (DM, H_LOC * DH))
    Wo_s = lax.dynamic_slice(Wo, (pos * H_LOC * DH, 0), (H_LOC * DH, DM))

    Q = (x[0] @ Wq_s).reshape(SQ, H_LOC, DH)

    def grp(t):
        return (
            t.reshape(8, 4, 64, H_LOC, DH)
            .transpose(1, 3, 0, 2, 4)
            .reshape(4, H_LOC, 512, DH)
        )

    ctxg = _attention(grp(Q), grp(K_ext[0]), grp(V_ext[0]))
    ctx = (
        ctxg.reshape(4, H_LOC, 8, 64, DH)
        .transpose(2, 0, 3, 1, 4)
        .reshape(SQ, H_LOC * DH)
    )

    outR_rot, outL_rot = _outproj_allreduce(ctx, Wo_s)
    outR = jnp.roll(outR_rot, shift=pos + 1, axis=0)
    outL = jnp.roll(outL_rot, shift=-pos, axis=0)[::-1]
    out = jnp.concatenate([outR, outL], axis=2).reshape(SQ, DM)
    return out[None]


# device time: 177707 ns/iter; 1.5400x vs baseline; 1.0125x over previous
import jax
import jax.numpy as jnp
from jax import lax
from jax.experimental import pallas as pl
from jax.experimental.pallas import tpu as pltpu

N_DEV = 8
SQ = 2048
DM = 1024
HD = DM // 2
H_LOC = 8
DH = 128
CHUNK = SQ // N_DEV
SCALE = 0.08838834764831843


def _outproj_allreduce(ctx, Wo_s):

    def body(ctx_ref, wo_ref, out_ref,
             bufR_ref, bufL_ref, commR_ref, commL_ref,
             send_sems, rs_sems, ag_sems):
        pos = lax.axis_index("i")
        right = lax.rem(pos + 1, N_DEV)
        left = lax.rem(pos + N_DEV - 1, N_DEV)

        def compute_chunk(m):
            c = lax.rem(pos - m + N_DEV, N_DEV)
            xr = ctx_ref[pl.ds(c * CHUNK, CHUNK), :]
            part = jnp.dot(xr, wo_ref[...],
                           preferred_element_type=jnp.float32)
            bufR_ref[(7 - m) % N_DEV] = part[:, :HD]
            bufL_ref[(m - 1) % N_DEV] = part[:, HD:]

        barrier = pltpu.get_barrier_semaphore()
        for nbr in (left, right):
            pl.semaphore_signal(
                barrier,
                inc=1,
                device_id=(nbr,),
                device_id_type=pl.DeviceIdType.MESH,
            )
        pl.semaphore_wait(barrier, 2)

        compute_chunk(0)

        def rs_hop(s, out_ref, comm_ref, dev, h):
            return pltpu.make_async_remote_copy(
                src_ref=out_ref.at[7 - s],
                dst_ref=comm_ref.at[6 - s],
                send_sem=send_sems.at[s, h, 0],
                recv_sem=rs_sems.at[s, h],
                device_id=(dev,),
                device_id_type=pl.DeviceIdType.MESH,
            )

        for s in range(N_DEV - 1):
            rdR = rs_hop(s, bufR_ref, commR_ref, right, 0)
            rdL = rs_hop(s, bufL_ref, commL_ref, left, 1)
            rdR.start()
            rdL.start()
            if s <= 2:
                compute_chunk(s + 1)
                compute_chunk(7 - s)
            elif s == 3:
                compute_chunk(4)
            rdR.wait()
            rdL.wait()
            bufR_ref[6 - s] = bufR_ref[6 - s] + commR_ref[6 - s]
            bufL_ref[6 - s] = bufL_ref[6 - s] + commL_ref[6 - s]

        def out_rows(c, col0):
            return out_ref.at[pl.ds(c * CHUNK, CHUNK), pl.ds(col0, HD)]

        def ag_hop(s, c, src, col0, dev, h):
            return pltpu.make_async_remote_copy(
                src_ref=src,
                dst_ref=out_rows(c, col0),
                send_sem=send_sems.at[s, h, 1],
                recv_sem=ag_sems.at[s, h],
                device_id=(dev,),
                device_id_type=pl.DeviceIdType.MESH,
            )

        for s in range(N_DEV - 1):
            cR = lax.rem(pos + 1 - s + N_DEV, N_DEV)
            cL = lax.rem(pos - 1 + s + N_DEV, N_DEV)
            srcR = bufR_ref.at[0] if s == 0 else out_rows(cR, 0)
            srcL = bufL_ref.at[0] if s == 0 else out_rows(cL, HD)
            rdR = ag_hop(s, cR, srcR, 0, right, 0)
            rdL = ag_hop(s, cL, srcL, HD, left, 1)
            rdR.start()
            rdL.start()
            if s == 0:
                cR0 = lax.rem(pos + 1, N_DEV)
                cL0 = lax.rem(pos - 1 + N_DEV, N_DEV)
                out_ref[pl.ds(cR0 * CHUNK, CHUNK), :HD] = bufR_ref[0]
                out_ref[pl.ds(cL0 * CHUNK, CHUNK), HD:] = bufL_ref[0]
            rdR.wait()
            rdL.wait()

    vmem = pl.BlockSpec(memory_space=pltpu.VMEM)
    return pl.pallas_call(
        body,
        out_shape=jax.ShapeDtypeStruct((SQ, DM), jnp.float32),
        in_specs=[vmem, vmem],
        out_specs=vmem,
        scratch_shapes=[
            pltpu.VMEM((N_DEV, CHUNK, HD), jnp.float32),
            pltpu.VMEM((N_DEV, CHUNK, HD), jnp.float32),
            pltpu.VMEM((N_DEV, CHUNK, HD), jnp.float32),
            pltpu.VMEM((N_DEV, CHUNK, HD), jnp.float32),
            pltpu.SemaphoreType.DMA((N_DEV - 1, 2, 2)),
            pltpu.SemaphoreType.DMA((N_DEV - 1, 2)),
            pltpu.SemaphoreType.DMA((N_DEV - 1, 2)),
        ],
        compiler_params=pltpu.CompilerParams(collective_id=0),
    )(ctx, Wo_s)


def _attention(Qt, Kt, Vt):

    def body(q_ref, k_ref, v_ref, o_ref):
        q = q_ref[0]
        k = k_ref[0]
        s = jax.lax.dot_general(
            q, k, (((2,), (2,)), ((0,), (0,))),
            preferred_element_type=jnp.float32,
        ) * SCALE
        s = s - jnp.max(s, axis=-1, keepdims=True)
        w = jnp.exp(s)
        w = w / jnp.sum(w, axis=-1, keepdims=True)
        o_ref[0] = jax.lax.dot_general(
            w, v_ref[0], (((2,), (1,)), ((0,), (0,))),
            preferred_element_type=jnp.float32,
        )

    blk = pl.BlockSpec((1, H_LOC, 512, DH), lambda g: (g, 0, 0, 0))
    return pl.pallas_call(
        body,
        grid=(4,),
        out_shape=jax.ShapeDtypeStruct((4, H_LOC, 512, DH), jnp.float32),
        in_specs=[blk, blk, blk],
        out_specs=blk,
    )(Qt, Kt, Vt)


def kernel(x, Wq, K_ext, V_ext, Wo):
    pos = lax.axis_index("i")

    Wq_s = lax.dynamic_slice(Wq, (0, pos * H_LOC * DH), (DM, H_LOC * DH))
    Wo_s = lax.dynamic_slice(Wo, (pos * H_LOC * DH, 0), (H_LOC * DH, DM))

    Q = (x[0] @ Wq_s).reshape(SQ, H_LOC, DH)

    def grp(t):
        return (
            t.reshape(8, 4, 64, H_LOC, DH)
            .transpose(1, 3, 0, 2, 4)
            .reshape(4, H_LOC, 512, DH)
        )

    ctxg = _attention(grp(Q), grp(K_ext[0]), grp(V_ext[0]))
    ctx = (
        ctxg.reshape(4, H_LOC, 8, 64, DH)
        .transpose(2, 0, 3, 1, 4)
        .reshape(SQ, H_LOC * DH)
    )

    out = _outproj_allreduce(ctx, Wo_s)
    return out[None]
